# baseline (speedup 1.0000x reference)
import jax
import jax.numpy as jnp
from jax.experimental import pallas as pl
from jax.experimental.pallas import tpu as pltpu

_MAX_LEN = 8192
_HIDDEN = 128
_NCH = 8
_CH = _MAX_LEN // _NCH


def _body(in_ref, out_ref, buf, rsems, wsems):
    reads = []
    for i in range(_NCH):
        c = pltpu.make_async_copy(
            in_ref.at[pl.ds(i * _CH, _CH)],
            buf.at[pl.ds(i * _CH, _CH)], rsems.at[i])
        c.start()
        reads.append(c)
    writes = []
    for i in range(_NCH):
        reads[i].wait()
        c = pltpu.make_async_copy(
            buf.at[pl.ds(i * _CH, _CH)],
            out_ref.at[pl.ds(i * _CH, _CH)], wsems.at[i])
        c.start()
        writes.append(c)
    for c in writes:
        c.wait()


def kernel(x, emb_table):
    seq_len = x.shape[1]
    out = pl.pallas_call(
        _body,
        in_specs=[pl.BlockSpec(memory_space=pl.ANY)],
        out_specs=pl.BlockSpec(memory_space=pl.ANY),
        scratch_shapes=[
            pltpu.VMEM((_MAX_LEN, _HIDDEN), jnp.float32),
            pltpu.SemaphoreType.DMA((_NCH,)),
            pltpu.SemaphoreType.DMA((_NCH,)),
        ],
        out_shape=jax.ShapeDtypeStruct((_MAX_LEN, _HIDDEN), jnp.float32),
    )(emb_table)
    return out[None, :seq_len]


# speedup vs baseline: 1.0047x; 1.0047x over previous
import jax
import jax.numpy as jnp
from jax.experimental import pallas as pl
from jax.experimental.pallas import tpu as pltpu

_MAX_LEN = 8192
_HIDDEN = 128
_NCH = 4
_CH = _MAX_LEN // _NCH


def _body(in_ref, out_ref, buf, rsems, wsems):
    reads = []
    for i in range(_NCH):
        c = pltpu.make_async_copy(
            in_ref.at[pl.ds(i * _CH, _CH)],
            buf.at[pl.ds(i * _CH, _CH)], rsems.at[i])
        c.start()
        reads.append(c)
    writes = []
    for i in range(_NCH):
        reads[i].wait()
        c = pltpu.make_async_copy(
            buf.at[pl.ds(i * _CH, _CH)],
            out_ref.at[pl.ds(i * _CH, _CH)], wsems.at[i])
        c.start()
        writes.append(c)
    for c in writes:
        c.wait()


def kernel(x, emb_table):
    seq_len = x.shape[1]
    out = pl.pallas_call(
        _body,
        in_specs=[pl.BlockSpec(memory_space=pl.ANY)],
        out_specs=pl.BlockSpec(memory_space=pl.ANY),
        scratch_shapes=[
            pltpu.VMEM((_MAX_LEN, _HIDDEN), jnp.float32),
            pltpu.SemaphoreType.DMA((_NCH,)),
            pltpu.SemaphoreType.DMA((_NCH,)),
        ],
        out_shape=jax.ShapeDtypeStruct((_MAX_LEN, _HIDDEN), jnp.float32),
    )(emb_table)
    return out[None, :seq_len]
